# Initial kernel scaffold; baseline (speedup 1.0000x reference)
import jax
import jax.numpy as jnp
from jax.experimental import pallas as pl


def kernel(inputs, target, negative_samples, emb_table):
    raise NotImplementedError("write your pallas kernel here")


# speedup vs baseline: 2.2977x; 2.2977x over previous
import functools

import jax
import jax.numpy as jnp
from jax import lax
from jax.experimental import pallas as pl
from jax.experimental.pallas import tpu as pltpu
from jax.experimental.pallas import tpu_sc as plsc

VOCAB = 1000000
EMBED = 64
BATCH = 16384
CTX = 20
NEG = 5
NSC = 6
R = CTX + 1 + NEG
NW = 32
CB = 32
PER_W = BATCH // NW
NCHUNK = PER_W // CB
NVD = EMBED // 16

VB = 16384
HALF = 507904
TGRID = HALF // VB


def _tr_body(x1_ref, x2_ref, y_ref):
    y_ref[:, 0:EMBED] = x1_ref[...].T
    y_ref[:, EMBED:2 * EMBED] = x2_ref[...].T


def _transpose_table(table_t):
    return pl.pallas_call(
        _tr_body,
        grid=(TGRID,),
        in_specs=[
            pl.BlockSpec((EMBED, VB), lambda i: (0, i)),
            pl.BlockSpec(
                (EMBED, VB), lambda i: (0, jnp.minimum(i + TGRID, VOCAB // VB))
            ),
        ],
        out_specs=pl.BlockSpec((VB, 2 * EMBED), lambda i: (i, 0)),
        out_shape=jax.ShapeDtypeStruct((HALF, 2 * EMBED), jnp.float32),
    )(table_t, table_t)


def _sc_body(idx_hbm, table_hbm, out_hbm, idx_v, rows_v, score_v, sem):
    wid = lax.axis_index("s") * 2 + lax.axis_index("c")

    def chunk_body(k, carry):
        pltpu.sync_copy(idx_hbm.at[wid, k], idx_v)
        copies = []
        for c in range(R):
            copies.append(
                pltpu.async_copy(table_hbm.at[idx_v.at[c]], rows_v.at[c], sem)
            )
        for cp in copies:
            cp.wait()

        lanes = lax.iota(jnp.int32, 16)
        zero = jnp.zeros((16,), jnp.float32)
        for g in range(CB // 16):

            def elem_body(el, carry2):
                e = g * 16 + el
                acc = [rows_v[0, e, pl.ds(16 * d, 16)] for d in range(NVD)]
                for c in range(1, CTX):
                    for d in range(NVD):
                        acc[d] = acc[d] + rows_v[c, e, pl.ds(16 * d, 16)]
                emb = [a * (1.0 / CTX) for a in acc]
                here = lanes == el
                out2 = []
                for j in range(NSC):
                    p = emb[0] * rows_v[CTX + j, e, pl.ds(0, 16)]
                    for d in range(1, NVD):
                        p = p + emb[d] * rows_v[CTX + j, e, pl.ds(16 * d, 16)]
                    out2.append(jnp.where(here, jnp.sum(p), carry2[j]))
                return tuple(out2)

            svecs = lax.fori_loop(0, 16, elem_body, (zero,) * NSC)
            for j in range(NSC):
                score_v[j, pl.ds(g * 16, 16)] = svecs[j]

        pltpu.sync_copy(score_v, out_hbm.at[wid, k])
        return carry

    lax.fori_loop(0, NCHUNK, chunk_body, 0)


@jax.jit
def _run(idx_arr, table_t):
    table_lin = _transpose_table(table_t).reshape(2 * HALF, EMBED)
    mesh = plsc.VectorSubcoreMesh(core_axis_name="c", subcore_axis_name="s")
    f = functools.partial(
        pl.kernel,
        out_type=jax.ShapeDtypeStruct((NW, NCHUNK, NSC, CB), jnp.float32),
        mesh=mesh,
        compiler_params=pltpu.CompilerParams(
            needs_layout_passes=False, use_tc_tiling_on_sc=False
        ),
        scratch_types=[
            pltpu.VMEM((R, CB), jnp.int32),
            pltpu.VMEM((R, CB, EMBED), jnp.float32),
            pltpu.VMEM((NSC, CB), jnp.float32),
            pltpu.SemaphoreType.DMA,
        ],
    )(_sc_body)
    return f(idx_arr, table_lin)


def kernel(inputs, target, negative_samples, emb_table):
    idx = jnp.concatenate(
        [inputs, target[:, None], negative_samples], axis=1
    ).astype(jnp.int32)
    idx = jnp.where(idx < HALF, 2 * idx, 2 * (idx - HALF) + 1)
    idx = idx.reshape(NW, NCHUNK, CB, R).transpose(0, 1, 3, 2)
    out = _run(idx, emb_table.T)
    out = out.transpose(0, 1, 3, 2).reshape(BATCH, NSC)
    return (out[:, 0], out[:, 1:])


# speedup vs baseline: 2.5592x; 1.1139x over previous
import functools

import jax
import jax.numpy as jnp
from jax import lax
from jax.experimental import pallas as pl
from jax.experimental.pallas import tpu as pltpu
from jax.experimental.pallas import tpu_sc as plsc

VOCAB = 1000000
EMBED = 64
BATCH = 16384
CTX = 20
NEG = 5
NSC = 6
R = CTX + 1 + NEG
NW = 32
CB = 32
PER_W = BATCH // NW
NCHUNK = PER_W // CB
NVD = EMBED // 16

VB = 16384
HALF = 507904
TGRID = HALF // VB


def _tr_body(x1_ref, x2_ref, y_ref):
    y_ref[:, 0:EMBED] = x1_ref[...].T
    y_ref[:, EMBED:2 * EMBED] = x2_ref[...].T


def _transpose_table(table_t):
    return pl.pallas_call(
        _tr_body,
        grid=(TGRID,),
        in_specs=[
            pl.BlockSpec((EMBED, VB), lambda i: (0, i)),
            pl.BlockSpec(
                (EMBED, VB), lambda i: (0, jnp.minimum(i + TGRID, VOCAB // VB))
            ),
        ],
        out_specs=pl.BlockSpec((VB, 2 * EMBED), lambda i: (i, 0)),
        out_shape=jax.ShapeDtypeStruct((HALF, 2 * EMBED), jnp.float32),
    )(table_t, table_t)


def _sc_body(idx_hbm, table_hbm, out_hbm, idx_v, rows0_v, rows1_v, score_v,
             sem0, sem1):
    wid = lax.axis_index("s") * 2 + lax.axis_index("c")
    pltpu.sync_copy(idx_hbm.at[wid], idx_v)

    def fire(k, rows_v, sem):
        return [
            pltpu.async_copy(table_hbm.at[idx_v.at[k, c]], rows_v.at[c], sem)
            for c in range(R)
        ]

    def drain(rows_v, sem):
        for c in range(R):
            pltpu.make_async_copy(table_hbm.at[idx_v.at[0, c]], rows_v.at[c],
                                  sem).wait()

    def compute(k, rows_v):
        lanes = lax.iota(jnp.int32, 16)
        zero = jnp.zeros((16,), jnp.float32)
        for g in range(CB // 16):

            def elem_body(el, carry2):
                e = g * 16 + el
                acc = [rows_v[0, e, pl.ds(16 * d, 16)] for d in range(NVD)]
                for c in range(1, CTX):
                    for d in range(NVD):
                        acc[d] = acc[d] + rows_v[c, e, pl.ds(16 * d, 16)]
                emb = [a * (1.0 / CTX) for a in acc]
                here = lanes == el
                out2 = []
                for j in range(NSC):
                    p = emb[0] * rows_v[CTX + j, e, pl.ds(0, 16)]
                    for d in range(1, NVD):
                        p = p + emb[d] * rows_v[CTX + j, e, pl.ds(16 * d, 16)]
                    out2.append(jnp.where(here, jnp.sum(p), carry2[j]))
                return tuple(out2)

            svecs = lax.fori_loop(0, 16, elem_body, (zero,) * NSC)
            for j in range(NSC):
                score_v[k, j, pl.ds(g * 16, 16)] = svecs[j]

    fire(0, rows0_v, sem0)

    def pair_body(h, carry):
        k0 = 2 * h
        drain(rows0_v, sem0)
        fire(k0 + 1, rows1_v, sem1)
        compute(k0, rows0_v)
        drain(rows1_v, sem1)
        fire(jnp.minimum(k0 + 2, NCHUNK - 1), rows0_v, sem0)
        compute(k0 + 1, rows1_v)
        return carry

    lax.fori_loop(0, NCHUNK // 2, pair_body, 0)
    drain(rows0_v, sem0)
    pltpu.sync_copy(score_v, out_hbm.at[wid])


@jax.jit
def _run(idx_arr, table_t):
    table_lin = _transpose_table(table_t).reshape(2 * HALF, EMBED)
    mesh = plsc.VectorSubcoreMesh(core_axis_name="c", subcore_axis_name="s")
    f = functools.partial(
        pl.kernel,
        out_type=jax.ShapeDtypeStruct((NW, NCHUNK, NSC, CB), jnp.float32),
        mesh=mesh,
        compiler_params=pltpu.CompilerParams(
            needs_layout_passes=False, use_tc_tiling_on_sc=False
        ),
        scratch_types=[
            pltpu.VMEM((NCHUNK, R, CB), jnp.int32),
            pltpu.VMEM((R, CB, EMBED), jnp.float32),
            pltpu.VMEM((R, CB, EMBED), jnp.float32),
            pltpu.VMEM((NCHUNK, NSC, CB), jnp.float32),
            pltpu.SemaphoreType.DMA,
            pltpu.SemaphoreType.DMA,
        ],
    )(_sc_body)
    return f(idx_arr, table_lin)


def kernel(inputs, target, negative_samples, emb_table):
    idx = jnp.concatenate(
        [inputs, target[:, None], negative_samples], axis=1
    ).astype(jnp.int32)
    idx = jnp.where(idx < HALF, 2 * idx, 2 * (idx - HALF) + 1)
    idx = idx.reshape(NW, NCHUNK, CB, R).transpose(0, 1, 3, 2)
    out = _run(idx, emb_table.T)
    out = out.transpose(0, 1, 3, 2).reshape(BATCH, NSC)
    return (out[:, 0], out[:, 1:])
